# baseline (speedup 1.0000x reference)
import functools

import jax
import jax.numpy as jnp
from jax.experimental import pallas as pl
from jax.experimental.pallas import tpu as pltpu

_NPF = 256
_NPX = 26
_NPY = 230


def _pos_kernel(b, h, w, ce_ref, re_ref, out_ref, pos, sems):
    ce_t = ce_ref[:].T
    re_t = re_ref[:].T
    pos[0:_NPX] = jnp.broadcast_to(ce_t[:, None, :], (_NPX, h, w))
    pos[_NPX:_NPF] = jnp.broadcast_to(re_t[:, :, None], (_NPY, h, w))
    for i in range(b):
        pltpu.make_async_copy(pos, out_ref.at[i], sems.at[i]).start()
    for i in range(b):
        pltpu.make_async_copy(pos, out_ref.at[i], sems.at[i]).wait()


def kernel(x, row_embed, col_embed):
    b = x.shape[0]
    h, w = x.shape[-2], x.shape[-1]
    ce = col_embed[:w]
    re = row_embed[:h]
    return pl.pallas_call(
        functools.partial(_pos_kernel, b, h, w),
        in_specs=[
            pl.BlockSpec(memory_space=pltpu.VMEM),
            pl.BlockSpec(memory_space=pltpu.VMEM),
        ],
        out_specs=pl.BlockSpec(memory_space=pl.ANY),
        out_shape=jax.ShapeDtypeStruct((b, _NPF, h, w), jnp.float32),
        scratch_shapes=[
            pltpu.VMEM((_NPF, h, w), jnp.float32),
            pltpu.SemaphoreType.DMA((b,)),
        ],
    )(ce, re)


# speedup vs baseline: 5.6690x; 5.6690x over previous
import functools

import jax
import jax.numpy as jnp
from jax.experimental import pallas as pl
from jax.experimental.pallas import tpu as pltpu

_NPF = 256
_NPX = 26
_NPY = 230


def _pos_kernel(b, h, w, top_ref, rep_ref, out_ref, pos, sems):
    pos[...] = (jnp.broadcast_to(top_ref[:][None, :, :], (h, w, _NPF))
                + jnp.broadcast_to(rep_ref[:][:, None, :], (h, w, _NPF)))
    for i in range(b):
        pltpu.make_async_copy(pos, out_ref.at[i], sems.at[i]).start()
    for i in range(b):
        pltpu.make_async_copy(pos, out_ref.at[i], sems.at[i]).wait()


def kernel(x, row_embed, col_embed):
    b = x.shape[0]
    h, w = x.shape[-2], x.shape[-1]
    ce = col_embed[:w]
    re = row_embed[:h]
    top = jnp.pad(ce, ((0, 0), (0, _NPY)))
    rep = jnp.pad(re, ((0, 0), (_NPX, 0)))
    out = pl.pallas_call(
        functools.partial(_pos_kernel, b, h, w),
        in_specs=[
            pl.BlockSpec(memory_space=pltpu.VMEM),
            pl.BlockSpec(memory_space=pltpu.VMEM),
        ],
        out_specs=pl.BlockSpec(memory_space=pl.ANY),
        out_shape=jax.ShapeDtypeStruct((b, h, w, _NPF), jnp.float32),
        scratch_shapes=[
            pltpu.VMEM((h, w, _NPF), jnp.float32),
            pltpu.SemaphoreType.DMA((b,)),
        ],
    )(top, rep)
    return jnp.transpose(out, (0, 3, 1, 2))
